# baseline (speedup 1.0000x reference)
import functools

import jax
import jax.numpy as jnp
from jax import lax
from jax.experimental import pallas as pl
from jax.experimental.pallas import tpu as pltpu
from jax.experimental.pallas import tpu_sc as plsc

C = 26
V = 100000
D = 32
B = 16384
W = 16
OUT = C * D + W
EPS = 1e-5

_L = 16
_BW = 512
_RPB = _BW // 4
_NFULL = V // _BW
_V128 = _NFULL * _BW
_VT = V - 32
_CB = 128
_RPF = V // 4
_NROWS = C * _RPF
_FPC = C // 2


def _rsqrt(x):
    xi = plsc.bitcast(x, jnp.int32)
    yi = jnp.int32(0x5F3759DF) - lax.shift_right_logical(xi, 1)
    y = plsc.bitcast(yi, jnp.float32)
    for _ in range(3):
        y = y * (1.5 - 0.5 * x * y * y)
    return y


def _body(deep_hbm, wide_hbm, tabt_hbm, tail_hbm, lnw_hbm, lnb_hbm,
          out_hbm, rows_hbm,
          tin_a, tin_b, tout_a, tout_b, land_a, land_b, slab_v,
          idx_a, idx_b, gv_a, gv_b, tail_v, wideb_v, wout_v, lnw_v, lnb_v,
          sem_ta, sem_tb, sem_ga, sem_gb):
    sc = lax.axis_index("c")
    t = lax.axis_index("s")
    iota = lax.broadcasted_iota(jnp.int32, (_L,), 0)
    tin = (tin_a, tin_b)
    tout = (tout_a, tout_b)
    land = (land_a, land_b)
    idxv = (idx_a, idx_b)
    gv = (gv_a, gv_b)
    sem_t = (sem_ta, sem_tb)
    sem_g = (sem_ga, sem_gb)

    def transpose_block(buf):
        def tr(r, _):
            for m in range(8):
                ll = m * _L + iota
                y = plsc.load_gather(
                    tin[buf], [ll & 31, lax.shift_right_logical(ll, 5) + 4 * r])
                plsc.store_scatter(tout[buf],
                                   [jnp.full((_L,), 0, jnp.int32) + r, ll], y)
            return 0
        lax.fori_loop(0, _RPB, tr, 0)

    def t_load(buf, c, slot):
        j = jnp.minimum(t + 16 * slot, _NFULL - 1)
        pltpu.sync_copy(tabt_hbm.at[c, :, pl.ds(j * _BW, _BW)], tin[buf])
        return j

    def t_store(buf, c, j):
        g0 = c * _RPF + j * _RPB
        return pltpu.async_copy(tout[buf], rows_hbm.at[pl.ds(g0, _RPB)],
                                sem_t[buf])

    def t_drain(buf):
        pltpu.make_async_copy(rows_hbm.at[pl.ds(0, _RPB)], tout[buf],
                              sem_t[buf]).wait()

    def g_fire(buf, c, k):
        base = c * B + t * 1024 + k * _CB
        pltpu.sync_copy(deep_hbm.at[pl.ds(base, _CB)], idxv[buf])
        for m in range(8):
            v = idxv[buf][pl.ds(m * _L, _L)]
            gv[buf][pl.ds(m * _L, _L)] = (
                c * _RPF + lax.shift_right_logical(v, 2))
        return pltpu.async_copy(rows_hbm.at[gv[buf]], land[buf], sem_g[buf])

    def g_drain(buf):
        pltpu.make_async_copy(rows_hbm.at[pl.ds(0, _CB)], land[buf],
                              sem_g[buf]).wait()

    def g_select(buf, c, k):
        jbs = []
        for m in range(8):
            v = idxv[buf][pl.ds(m * _L, _L)]
            jbs.append(lax.shift_left(v & 3, 5))

        def dim(d, _):
            for m in range(8):
                y = plsc.load_gather(land[buf], [m * _L + iota, jbs[m] + d])
                plsc.store_scatter(
                    slab_v, [jnp.full((_L,), 0, jnp.int32) + d, m * _L + iota],
                    y)
            return 0
        lax.fori_loop(0, D, dim, 0)
        pltpu.sync_copy(
            slab_v, out_hbm.at[pl.ds(c * D, D),
                               pl.ds(t * 1024 + k * _CB, _CB)])

    def field(ci, _):
        c = sc * _FPC + ci

        j0 = t_load(0, c, 0)
        transpose_block(0)
        t_store(0, c, j0)

        def tloop(i, _):
            jb_ = t_load(1, c, 2 * i + 1)

            @pl.when(i > 0)
            def _():
                t_drain(1)
            transpose_block(1)
            t_store(1, c, jb_)
            ja_ = t_load(0, c, 2 * i + 2)
            t_drain(0)
            transpose_block(0)
            t_store(0, c, ja_)
            return 0

        lax.fori_loop(0, 6, tloop, 0)
        t_drain(0)
        t_drain(1)

        @pl.when(t == 2)
        def _():
            pltpu.sync_copy(tabt_hbm.at[c, :, pl.ds(_V128, 128)],
                            tin[0].at[:, pl.ds(0, 128)])
            def tr(r, _):
                for m in range(8):
                    ll = m * _L + iota
                    y = plsc.load_gather(
                        tin[0], [ll & 31, lax.shift_right_logical(ll, 5) + 4 * r])
                    plsc.store_scatter(
                        tout[0], [jnp.full((_L,), 0, jnp.int32) + r, ll], y)
                return 0
            lax.fori_loop(0, 32, tr, 0)
            pltpu.sync_copy(tout[0].at[pl.ds(0, 32)],
                            rows_hbm.at[pl.ds(c * _RPF + _V128 // 4, 32)])

        @pl.when(t == 3)
        def _():
            pltpu.sync_copy(tail_hbm.at[pl.ds(c * 1024, 1024)], tail_v)
            def tr(r, _):
                for m in range(8):
                    ll = m * _L + iota
                    y = plsc.load_gather(
                        tail_v,
                        [lax.shift_left(ll & 31, 5)
                         + lax.shift_right_logical(ll, 5) + 4 * r])
                    plsc.store_scatter(
                        tout[1], [jnp.full((_L,), 0, jnp.int32) + r, ll], y)
                return 0
            lax.fori_loop(0, 8, tr, 0)
            pltpu.sync_copy(tout[1].at[pl.ds(0, 8)],
                            rows_hbm.at[pl.ds(c * _RPF + _VT // 4, 8)])

        plsc.subcore_barrier()

        g_fire(0, c, 0)

        def gloop(i, _):
            g_fire(1, c, 2 * i + 1)
            g_drain(0)
            g_select(0, c, 2 * i)

            @pl.when(i < 3)
            def _():
                g_fire(0, c, 2 * i + 2)
            g_drain(1)
            g_select(1, c, 2 * i + 1)
            return 0
        lax.fori_loop(0, 4, gloop, 0)
        return 0

    lax.fori_loop(0, _FPC, field, 0)

    @pl.when(sc == 0)
    def _():
        pltpu.sync_copy(lnw_hbm, lnw_v)
        pltpu.sync_copy(lnb_hbm, lnb_v)

        def wchunk(k, _):
            pltpu.sync_copy(
                wide_hbm.at[:, pl.ds(t * 1024 + k * _CB, _CB)], wideb_v)

            def grp(m, _):
                xs = [wideb_v[f, pl.ds(m * _L, _L)] for f in range(W)]
                s = xs[0]
                for f in range(1, W):
                    s = s + xs[f]
                mean = s * (1.0 / W)
                var = (xs[0] - mean) * (xs[0] - mean)
                for f in range(1, W):
                    var = var + (xs[f] - mean) * (xs[f] - mean)
                r = _rsqrt(var * (1.0 / W) + EPS)
                for f in range(W):
                    lw = plsc.load_gather(lnw_v, [jnp.full((_L,), f, jnp.int32)])
                    lb = plsc.load_gather(lnb_v, [jnp.full((_L,), f, jnp.int32)])
                    wout_v[f, pl.ds(m * _L, _L)] = (xs[f] - mean) * r * lw + lb
                return 0
            lax.fori_loop(0, _CB // _L, grp, 0)
            pltpu.sync_copy(
                wout_v,
                out_hbm.at[pl.ds(C * D, W), pl.ds(t * 1024 + k * _CB, _CB)])
            return 0
        lax.fori_loop(0, B // 16 // _CB, wchunk, 0)


def kernel(deep_in, wide_in, tables, ln_w, ln_b):
    tables_t = jnp.transpose(tables, (0, 2, 1))
    tail = jnp.transpose(tables[:, _VT:, :], (0, 2, 1)).reshape(-1)
    deep1 = jnp.reshape(deep_in, (-1,))
    mesh = plsc.VectorSubcoreMesh(core_axis_name="c", subcore_axis_name="s")
    k = functools.partial(
        pl.kernel,
        mesh=mesh,
        compiler_params=pltpu.CompilerParams(needs_layout_passes=False),
        out_type=(jax.ShapeDtypeStruct((OUT, B), jnp.float32),
                  jax.ShapeDtypeStruct((_NROWS, 128), jnp.float32)),
        scratch_types=[
            pltpu.VMEM((D, _BW), jnp.float32),
            pltpu.VMEM((D, _BW), jnp.float32),
            pltpu.VMEM((_RPB, 128), jnp.float32),
            pltpu.VMEM((_RPB, 128), jnp.float32),
            pltpu.VMEM((_CB, 128), jnp.float32),
            pltpu.VMEM((_CB, 128), jnp.float32),
            pltpu.VMEM((D, _CB), jnp.float32),
            pltpu.VMEM((_CB,), jnp.int32),
            pltpu.VMEM((_CB,), jnp.int32),
            pltpu.VMEM((_CB,), jnp.int32),
            pltpu.VMEM((_CB,), jnp.int32),
            pltpu.VMEM((1024,), jnp.float32),
            pltpu.VMEM((W, _CB), jnp.float32),
            pltpu.VMEM((W, _CB), jnp.float32),
            pltpu.VMEM((W,), jnp.float32),
            pltpu.VMEM((W,), jnp.float32),
            pltpu.SemaphoreType.DMA,
            pltpu.SemaphoreType.DMA,
            pltpu.SemaphoreType.DMA,
            pltpu.SemaphoreType.DMA,
        ],
    )(_body)
    out_t, _ = k(deep1, wide_in, tables_t, tail, ln_w, ln_b)
    return jnp.transpose(out_t)


# speedup vs baseline: 1.4780x; 1.4780x over previous
import functools

import jax
import jax.numpy as jnp
from jax import lax
from jax.experimental import pallas as pl
from jax.experimental.pallas import tpu as pltpu
from jax.experimental.pallas import tpu_sc as plsc

C = 26
V = 100000
D = 32
B = 16384
W = 16
OUT = C * D + W
EPS = 1e-5

_CB = 128
_TCW = 512
_RPF = 32768
_NVB = (V + _TCW - 1) // _TCW - 1
_INFO = plsc.get_sparse_core_info()
_NC, _NS, _L = _INFO.num_cores, _INFO.num_subcores, _INFO.num_lanes
_NW = _NC * _NS
_BPW = B // _NW
_NCHUNK = _BPW // _CB


def _rsqrt(x):
    xi = plsc.bitcast(x, jnp.int32)
    yi = jnp.int32(0x5F3759DF) - lax.shift_right_logical(xi, 1)
    y = plsc.bitcast(yi, jnp.float32)
    for _ in range(3):
        y = y * (1.5 - 0.5 * x * y * y)
    return y


def _tc_transform_body(x0, x1, x2, x3, out_ref):
    for j, x in enumerate((x0, x1, x2, x3)):
        out_ref[:, pl.ds(j * D, D)] = jnp.transpose(x[0])


def _tc_transform(tables_t):
    nq = _RPF // _TCW
    specs = [
        pl.BlockSpec(
            (1, D, _TCW),
            functools.partial(
                lambda jj, c, q: (c, 0, jnp.minimum(nq * jj + q, _NVB)), j))
        for j in range(4)
    ]
    return pl.pallas_call(
        _tc_transform_body,
        grid=(C, nq),
        in_specs=specs,
        out_specs=pl.BlockSpec((_TCW, 128), lambda c, q: (c * nq + q, 0)),
        out_shape=jax.ShapeDtypeStruct((C * _RPF, 128), jnp.float32),
    )(tables_t, tables_t, tables_t, tables_t)


def _body(deep_hbm, wide_hbm, tabrows_hbm, lnw_hbm, lnb_hbm, out_hbm,
          idx_v, jb_v, wide_v, land_v, trans_v, wout_v, lnwb_v, sem):
    wid = lax.axis_index("s") * _NC + lax.axis_index("c")
    base0 = wid * _BPW
    iota = lax.broadcasted_iota(jnp.int32, (_L,), 0)

    pltpu.sync_copy(lnw_hbm, lnwb_v.at[0])
    pltpu.sync_copy(lnb_hbm, lnwb_v.at[1])

    pltpu.sync_copy(deep_hbm.at[:, pl.ds(base0, _BPW)], idx_v)
    pltpu.sync_copy(wide_hbm.at[:, pl.ds(base0, _BPW)], wide_v)

    for c in range(C):
        def cvt(k, _, c=c):
            v = idx_v[c, pl.ds(k * _L, _L)]
            idx_v[c, pl.ds(k * _L, _L)] = c * _RPF + (v & (_RPF - 1))
            jb_v[c, pl.ds(k * _L, _L)] = lax.shift_left(
                lax.shift_right_logical(v, 15), 5)
            return 0
        lax.fori_loop(0, _BPW // _L, cvt, 0)

    def chunk(k, _):
        base = base0 + k * _CB

        for c in range(C):
            cp = pltpu.async_copy(
                tabrows_hbm.at[idx_v.at[c, pl.ds(k * _CB, _CB)]], land_v, sem)
            cp.wait()

            def grp(k2, _, c=c):
                jb = jb_v[c, pl.ds(k * _CB + k2 * _L, _L)]
                rows = k2 * _L + iota

                def dim(d, _):
                    y = plsc.load_gather(land_v, [rows, jb + d])
                    plsc.store_scatter(
                        trans_v, [jnp.full((_L,), d, jnp.int32),
                                  k2 * _L + iota], y)
                    return 0
                lax.fori_loop(0, D, dim, 0)
                return 0
            lax.fori_loop(0, _CB // _L, grp, 0)
            pltpu.sync_copy(trans_v,
                            out_hbm.at[pl.ds(c * D, D), pl.ds(base, _CB)])

        def wgrp(k2, _):
            xs = [wide_v[f, pl.ds(k * _CB + k2 * _L, _L)] for f in range(W)]
            s = xs[0]
            for f in range(1, W):
                s = s + xs[f]
            mean = s * (1.0 / W)
            var = (xs[0] - mean) * (xs[0] - mean)
            for f in range(1, W):
                var = var + (xs[f] - mean) * (xs[f] - mean)
            r = _rsqrt(var * (1.0 / W) + EPS)
            for f in range(W):
                lw = plsc.load_gather(
                    lnwb_v, [jnp.full((_L,), 0, jnp.int32),
                             jnp.full((_L,), f, jnp.int32)])
                lb = plsc.load_gather(
                    lnwb_v, [jnp.full((_L,), 1, jnp.int32),
                             jnp.full((_L,), f, jnp.int32)])
                wout_v[f, pl.ds(k2 * _L, _L)] = (xs[f] - mean) * r * lw + lb
            return 0
        lax.fori_loop(0, _CB // _L, wgrp, 0)
        pltpu.sync_copy(wout_v, out_hbm.at[pl.ds(C * D, W), pl.ds(base, _CB)])
        return 0

    lax.fori_loop(0, _NCHUNK, chunk, 0)


def kernel(deep_in, wide_in, tables, ln_w, ln_b):
    tables_t = jnp.transpose(tables, (0, 2, 1))
    tabrows = _tc_transform(tables_t)
    mesh = plsc.VectorSubcoreMesh(core_axis_name="c", subcore_axis_name="s")
    k = functools.partial(
        pl.kernel,
        mesh=mesh,
        compiler_params=pltpu.CompilerParams(needs_layout_passes=False),
        out_type=jax.ShapeDtypeStruct((OUT, B), jnp.float32),
        scratch_types=[
            pltpu.VMEM((C, _BPW), jnp.int32),
            pltpu.VMEM((C, _BPW), jnp.int32),
            pltpu.VMEM((W, _BPW), jnp.float32),
            pltpu.VMEM((_CB, 128), jnp.float32),
            pltpu.VMEM((D, _CB), jnp.float32),
            pltpu.VMEM((W, _CB), jnp.float32),
            pltpu.VMEM((2, W), jnp.float32),
            pltpu.SemaphoreType.DMA,
        ],
    )(_body)
    out_t = k(deep_in, wide_in, tabrows, ln_w, ln_b)
    return jnp.transpose(out_t)


# speedup vs baseline: 2.6069x; 1.7638x over previous
import functools

import jax
import jax.numpy as jnp
from jax import lax
from jax.experimental import pallas as pl
from jax.experimental.pallas import tpu as pltpu
from jax.experimental.pallas import tpu_sc as plsc

C = 26
V = 100000
D = 32
B = 16384
W = 16
OUT = C * D + W
EPS = 1e-5

_CB = 128
_INFO = plsc.get_sparse_core_info()
_NC, _NS, _L = _INFO.num_cores, _INFO.num_subcores, _INFO.num_lanes
_NW = _NC * _NS
_BPW = B // _NW
_NCHUNK = _BPW // _CB


def _rsqrt(x):
    xi = plsc.bitcast(x, jnp.int32)
    yi = jnp.int32(0x5F3759DF) - lax.shift_right_logical(xi, 1)
    y = plsc.bitcast(yi, jnp.float32)
    for _ in range(3):
        y = y * (1.5 - 0.5 * x * y * y)
    return y


def _body(deep_hbm, wide_hbm, tflat_hbm, lnw_hbm, lnb_hbm, out_hbm,
          idx_v, wide_v, idx2_v, land_v, wout_v, lnwb_v, sem):
    wid = lax.axis_index("s") * _NC + lax.axis_index("c")
    base0 = wid * _BPW

    pltpu.sync_copy(lnw_hbm, lnwb_v.at[0])
    pltpu.sync_copy(lnb_hbm, lnwb_v.at[1])

    pltpu.sync_copy(deep_hbm.at[:, pl.ds(base0, _BPW)], idx_v)
    pltpu.sync_copy(wide_hbm.at[:, pl.ds(base0, _BPW)], wide_v)

    def chunk(k, _):
        base = base0 + k * _CB

        for c in range(C):
            def mkfire(d, _, c=c):
                off = d * V + c * (D * V)
                for m in range(_CB // _L):
                    v = idx_v[c, pl.ds(k * _CB + m * _L, _L)]
                    idx2_v[d, pl.ds(m * _L, _L)] = v + off
                pltpu.make_async_copy(
                    tflat_hbm.at[idx2_v.at[d]], land_v.at[d], sem).start()
                return 0
            lax.fori_loop(0, D, mkfire, 0)

            def drain(d, _):
                pltpu.make_async_copy(
                    tflat_hbm.at[idx2_v.at[d]], land_v.at[d], sem).wait()
                return 0
            lax.fori_loop(0, D, drain, 0)
            pltpu.sync_copy(land_v,
                            out_hbm.at[pl.ds(c * D, D), pl.ds(base, _CB)])

        def wgrp(k2, _):
            xs = [wide_v[f, pl.ds(k * _CB + k2 * _L, _L)] for f in range(W)]
            s = xs[0]
            for f in range(1, W):
                s = s + xs[f]
            mean = s * (1.0 / W)
            var = (xs[0] - mean) * (xs[0] - mean)
            for f in range(1, W):
                var = var + (xs[f] - mean) * (xs[f] - mean)
            r = _rsqrt(var * (1.0 / W) + EPS)
            for f in range(W):
                lw = plsc.load_gather(
                    lnwb_v, [jnp.full((_L,), 0, jnp.int32),
                             jnp.full((_L,), f, jnp.int32)])
                lb = plsc.load_gather(
                    lnwb_v, [jnp.full((_L,), 1, jnp.int32),
                             jnp.full((_L,), f, jnp.int32)])
                wout_v[f, pl.ds(k2 * _L, _L)] = (xs[f] - mean) * r * lw + lb
            return 0
        lax.fori_loop(0, _CB // _L, wgrp, 0)
        pltpu.sync_copy(wout_v, out_hbm.at[pl.ds(C * D, W), pl.ds(base, _CB)])
        return 0

    lax.fori_loop(0, _NCHUNK, chunk, 0)


def kernel(deep_in, wide_in, tables, ln_w, ln_b):
    tflat = jnp.reshape(jnp.transpose(tables, (0, 2, 1)), (-1,))
    mesh = plsc.VectorSubcoreMesh(core_axis_name="c", subcore_axis_name="s")
    k = functools.partial(
        pl.kernel,
        mesh=mesh,
        compiler_params=pltpu.CompilerParams(needs_layout_passes=False),
        out_type=jax.ShapeDtypeStruct((OUT, B), jnp.float32),
        scratch_types=[
            pltpu.VMEM((C, _BPW), jnp.int32),
            pltpu.VMEM((W, _BPW), jnp.float32),
            pltpu.VMEM((D, _CB), jnp.int32),
            pltpu.VMEM((D, _CB), jnp.float32),
            pltpu.VMEM((W, _CB), jnp.float32),
            pltpu.VMEM((2, W), jnp.float32),
            pltpu.SemaphoreType.DMA,
        ],
    )(_body)
    out_t = k(deep_in, wide_in, tflat, ln_w, ln_b)
    return jnp.transpose(out_t)


# speedup vs baseline: 2.8935x; 1.1099x over previous
import functools

import jax
import jax.numpy as jnp
from jax import lax
from jax.experimental import pallas as pl
from jax.experimental.pallas import tpu as pltpu
from jax.experimental.pallas import tpu_sc as plsc

C = 26
V = 100000
D = 32
B = 16384
W = 16
OUT = C * D + W
EPS = 1e-5

_CB = 128
_INFO = plsc.get_sparse_core_info()
_NC, _NS, _L = _INFO.num_cores, _INFO.num_subcores, _INFO.num_lanes
_NW = _NC * _NS
_BPW = B // _NW
_NCHUNK = _BPW // _CB


def _rsqrt(x):
    xi = plsc.bitcast(x, jnp.int32)
    yi = jnp.int32(0x5F3759DF) - lax.shift_right_logical(xi, 1)
    y = plsc.bitcast(yi, jnp.float32)
    for _ in range(3):
        y = y * (1.5 - 0.5 * x * y * y)
    return y


def _body(deep_hbm, wide_hbm, tflat_hbm, lnw_hbm, lnb_hbm, out_hbm,
          idx_v, wide_v, idx2_a, idx2_b, land_a, land_b, wout_v, lnwb_v,
          sem_a, sem_b):
    wid = lax.axis_index("s") * _NC + lax.axis_index("c")
    base0 = wid * _BPW

    pltpu.sync_copy(lnw_hbm, lnwb_v.at[0])
    pltpu.sync_copy(lnb_hbm, lnwb_v.at[1])

    pltpu.sync_copy(deep_hbm.at[:, pl.ds(base0, _BPW)], idx_v)
    pltpu.sync_copy(wide_hbm.at[:, pl.ds(base0, _BPW)], wide_v)

    def chunk(k, _):
        base = base0 + k * _CB

        idx2 = (idx2_a, idx2_b)
        land = (land_a, land_b)
        sems = (sem_a, sem_b)

        def fire(c):
            p = c % 2

            def mkfire(d, _, c=c, p=p):
                off = d * V + c * (D * V)
                for m in range(_CB // _L):
                    v = idx_v[c, pl.ds(k * _CB + m * _L, _L)]
                    idx2[p][d, pl.ds(m * _L, _L)] = v + off
                pltpu.make_async_copy(
                    tflat_hbm.at[idx2[p].at[d]], land[p].at[d],
                    sems[p]).start()
                return 0
            lax.fori_loop(0, D, mkfire, 0)

        def finish(c):
            p = c % 2

            def drain(d, _, p=p):
                pltpu.make_async_copy(
                    tflat_hbm.at[idx2[p].at[d]], land[p].at[d],
                    sems[p]).wait()
                return 0
            lax.fori_loop(0, D, drain, 0)
            pltpu.sync_copy(land[p],
                            out_hbm.at[pl.ds(c * D, D), pl.ds(base, _CB)])

        fire(0)
        for c in range(1, C):
            fire(c)
            finish(c - 1)
        finish(C - 1)

        def wgrp(k2, _):
            xs = [wide_v[f, pl.ds(k * _CB + k2 * _L, _L)] for f in range(W)]
            s = xs[0]
            for f in range(1, W):
                s = s + xs[f]
            mean = s * (1.0 / W)
            var = (xs[0] - mean) * (xs[0] - mean)
            for f in range(1, W):
                var = var + (xs[f] - mean) * (xs[f] - mean)
            r = _rsqrt(var * (1.0 / W) + EPS)
            for f in range(W):
                lw = plsc.load_gather(
                    lnwb_v, [jnp.full((_L,), 0, jnp.int32),
                             jnp.full((_L,), f, jnp.int32)])
                lb = plsc.load_gather(
                    lnwb_v, [jnp.full((_L,), 1, jnp.int32),
                             jnp.full((_L,), f, jnp.int32)])
                wout_v[f, pl.ds(k2 * _L, _L)] = (xs[f] - mean) * r * lw + lb
            return 0
        lax.fori_loop(0, _CB // _L, wgrp, 0)
        pltpu.sync_copy(wout_v, out_hbm.at[pl.ds(C * D, W), pl.ds(base, _CB)])
        return 0

    lax.fori_loop(0, _NCHUNK, chunk, 0)


def kernel(deep_in, wide_in, tables, ln_w, ln_b):
    tflat = jnp.reshape(jnp.transpose(tables, (0, 2, 1)), (-1,))
    mesh = plsc.VectorSubcoreMesh(core_axis_name="c", subcore_axis_name="s")
    k = functools.partial(
        pl.kernel,
        mesh=mesh,
        compiler_params=pltpu.CompilerParams(needs_layout_passes=False),
        out_type=jax.ShapeDtypeStruct((OUT, B), jnp.float32),
        scratch_types=[
            pltpu.VMEM((C, _BPW), jnp.int32),
            pltpu.VMEM((W, _BPW), jnp.float32),
            pltpu.VMEM((D, _CB), jnp.int32),
            pltpu.VMEM((D, _CB), jnp.int32),
            pltpu.VMEM((D, _CB), jnp.float32),
            pltpu.VMEM((D, _CB), jnp.float32),
            pltpu.VMEM((W, _CB), jnp.float32),
            pltpu.VMEM((2, W), jnp.float32),
            pltpu.SemaphoreType.DMA,
            pltpu.SemaphoreType.DMA,
        ],
    )(_body)
    out_t = k(deep_in, wide_in, tflat, ln_w, ln_b)
    return jnp.transpose(out_t)
